# Initial kernel scaffold; baseline (speedup 1.0000x reference)
import functools

import jax
import jax.numpy as jnp
from jax import lax
from jax.experimental import pallas as pl
from jax.experimental.pallas import tpu as pltpu
from jax.experimental.pallas import tpu_sc as plsc

D = 64
L = 20
NB = 20
NC = 2
NS = 16
NW = NC * NS
LANES = 16


def _sc_embed(item_id, genres_flat, xs, bounds_pad, item_table, gt_flat, t3_flat):
  B = item_id.shape[0]
  bpw = B // NW
  sub = 128
  n_sub = bpw // sub
  ig_chunk = 256

  mesh = plsc.VectorSubcoreMesh(core_axis_name="c", subcore_axis_name="s")

  @functools.partial(
      pl.kernel,
      mesh=mesh,
      out_type=(
          jax.ShapeDtypeStruct((B, D), jnp.float32),
          jax.ShapeDtypeStruct((B * D,), jnp.float32),
          jax.ShapeDtypeStruct((B * D,), jnp.float32),
      ),
      scratch_types=(
          pltpu.VMEM((ig_chunk,), jnp.int32),
          pltpu.VMEM((ig_chunk,), jnp.int32),
          pltpu.VMEM((ig_chunk, D), jnp.float32),
          pltpu.VMEM((bpw * L,), jnp.int32),
          pltpu.VMEM((1000 * D,), jnp.float32),
          pltpu.VMEM((3 * NB * D,), jnp.float32),
          pltpu.VMEM((32,), jnp.float32),
          pltpu.VMEM((3 * bpw,), jnp.float32),
          pltpu.VMEM((sub * D,), jnp.float32),
          pltpu.VMEM((sub * D,), jnp.float32),
          pltpu.SemaphoreType.DMA,
      ),
  )
  def k(item_id_h, genres_h, xs_h, bounds_h, table_h, gt_h, t3_h,
        iout_h, gout_h, cout_h,
        idx_a, idx_b, rows_v, gen_v, gt_v, t3_v, bnd_v, xs_v, gbuf, cbuf, sem):
    wid = lax.axis_index("s") * NC + lax.axis_index("c")
    base = wid * bpw

    pltpu.sync_copy(item_id_h.at[pl.ds(base, ig_chunk)], idx_a)
    cp_a = pltpu.async_copy(table_h.at[idx_a], rows_v, sem)
    pltpu.sync_copy(item_id_h.at[pl.ds(base + ig_chunk, ig_chunk)], idx_b)
    pltpu.sync_copy(genres_h.at[pl.ds(base * L, bpw * L)], gen_v)
    pltpu.sync_copy(gt_h, gt_v)
    pltpu.sync_copy(t3_h, t3_v)
    pltpu.sync_copy(bounds_h, bnd_v)
    for t in range(3):
      pltpu.sync_copy(xs_h.at[pl.ds(t * B + base, bpw)], xs_v.at[pl.ds(t * bpw, bpw)])

    zero = jnp.zeros((LANES,), jnp.float32)
    for j in range(D // LANES):
      gt_v[pl.ds(LANES * j, LANES)] = zero

    iota = lax.iota(jnp.int32, LANES)
    ba = bnd_v[pl.ds(0, LANES)]
    bb = bnd_v[pl.ds(LANES, LANES)]

    def row_body(i, sck):
      bi = sck * sub + i
      gbase = bi * L
      acc = [zero, zero, zero, zero]
      cnt = jnp.float32(0.0)
      for l in range(L):
        g = gen_v[gbase + l]
        cnt = cnt + jnp.where(g > 0, jnp.float32(1.0), jnp.float32(0.0))
        rb = g * D
        for j in range(D // LANES):
          idx = jnp.full((LANES,), rb + LANES * j, jnp.int32) + iota
          acc[j] = acc[j] + plsc.load_gather(gt_v, [idx])
      inv = jnp.float32(1.0) / (cnt + jnp.float32(1e-8))
      for j in range(D // LANES):
        gbuf[pl.ds(i * D + LANES * j, LANES)] = acc[j] * inv
      cacc = [zero, zero, zero, zero]
      for t in range(3):
        x = xs_v[t * bpw + bi]
        xsp = jnp.full((LANES,), x, jnp.float32)
        bidx = (plsc.all_reduce_population_count(ba < xsp)
                + plsc.all_reduce_population_count(bb < xsp))
        rb2 = bidx * D + (t * NB * D)
        for j in range(D // LANES):
          cacc[j] = cacc[j] + plsc.load_gather(t3_v, [rb2 + LANES * j + iota])
      for j in range(D // LANES):
        cbuf[pl.ds(i * D + LANES * j, LANES)] = cacc[j]
      return sck

    for sck in range(n_sub):
      lax.fori_loop(0, sub, row_body, sck)
      pltpu.sync_copy(gbuf, gout_h.at[pl.ds((base + sck * sub) * D, sub * D)])
      pltpu.sync_copy(cbuf, cout_h.at[pl.ds((base + sck * sub) * D, sub * D)])
      if sck == n_sub // 2 - 1:
        cp_a.wait()
        pltpu.sync_copy(rows_v, iout_h.at[pl.ds(base, ig_chunk)])
        cp_b = pltpu.async_copy(table_h.at[idx_b], rows_v, sem)
    cp_b.wait()
    pltpu.sync_copy(rows_v, iout_h.at[pl.ds(base + ig_chunk, ig_chunk)])

  return k(item_id, genres_flat, xs, bounds_pad, item_table, gt_flat, t3_flat)


def _mlp(i_emb, g_emb, c_emb, W1, bias3, W2, b2):
  B = i_emb.shape[0]
  blk = 2048

  def body(i_ref, g_ref, c_ref, w1_ref, bias_ref, w2_ref, b2_ref, o_ref):
    h = (jnp.dot(i_ref[...], w1_ref[0:D, :], preferred_element_type=jnp.float32)
         + jnp.dot(g_ref[...], w1_ref[D:2 * D, :], preferred_element_type=jnp.float32)
         + jnp.dot(c_ref[...], w1_ref[2 * D:3 * D, :], preferred_element_type=jnp.float32)
         + bias_ref[0:1, :])
    mu = jnp.mean(h, axis=1, keepdims=True)
    d = h - mu
    var = jnp.mean(d * d, axis=1, keepdims=True)
    hn = d * lax.rsqrt(var + 1e-5) * bias_ref[1:2, :] + bias_ref[2:3, :]
    hn = jnp.maximum(hn, 0.0)
    out = jnp.dot(hn, w2_ref[...], preferred_element_type=jnp.float32) + b2_ref[0:1, :]
    nrm = jnp.sqrt(jnp.sum(out * out, axis=1, keepdims=True))
    o_ref[...] = out / jnp.maximum(nrm, 1e-12)

  return pl.pallas_call(
      body,
      grid=(B // blk,),
      in_specs=[
          pl.BlockSpec((blk, D), lambda i: (i, 0)),
          pl.BlockSpec((blk, D), lambda i: (i, 0)),
          pl.BlockSpec((blk, D), lambda i: (i, 0)),
          pl.BlockSpec((3 * D, 2 * D), lambda i: (0, 0)),
          pl.BlockSpec((4, 2 * D), lambda i: (0, 0)),
          pl.BlockSpec((2 * D, D), lambda i: (0, 0)),
          pl.BlockSpec((1, D), lambda i: (0, 0)),
      ],
      out_specs=pl.BlockSpec((blk, D), lambda i: (i, 0)),
      out_shape=jax.ShapeDtypeStruct((B, D), jnp.float32),
  )(i_emb, g_emb, c_emb, W1, bias3, W2, b2)


def kernel(item_id, tmdb_genres, release_year, avg_rating, revenue,
           item_table, genre_table, ry_table, ar_table, rev_table, bounds,
           W1, b1, g1, be1, W2, b2):
  B = item_id.shape[0]
  genres_flat = tmdb_genres.reshape(-1).astype(jnp.int32)
  xs = jnp.concatenate([release_year, avg_rating, revenue])
  bounds_pad = jnp.concatenate(
      [bounds, jnp.full((32 - bounds.shape[0],), 2.0, jnp.float32)])
  gt_flat = genre_table.reshape(-1)
  t3_flat = jnp.concatenate(
      [ry_table.reshape(-1), ar_table.reshape(-1), rev_table.reshape(-1)])

  i_emb, g_flat, c_flat = _sc_embed(
      item_id.astype(jnp.int32), genres_flat, xs, bounds_pad,
      item_table, gt_flat, t3_flat)
  g_emb = g_flat.reshape(B, D)
  c_emb = c_flat.reshape(B, D)

  bias3 = jnp.concatenate(
      [b1[None, :], g1[None, :], be1[None, :],
       jnp.zeros((1, 2 * D), jnp.float32)], axis=0)
  return _mlp(i_emb, g_emb, c_emb, W1, bias3, W2, b2[None, :])


# speedup vs baseline: 2.9080x; 2.9080x over previous
import functools

import jax
import jax.numpy as jnp
from jax import lax
from jax.experimental import pallas as pl
from jax.experimental.pallas import tpu as pltpu
from jax.experimental.pallas import tpu_sc as plsc

D = 64
L = 20
NB = 20
NC = 2
NS = 16
NW = NC * NS
LANES = 16


def _sc_embed(item_id, genres_flat, xs, bounds_pad, item_table, gt_flat, t3_flat):
  B = item_id.shape[0]
  bpw = B // NW
  sub = 128
  n_sub = bpw // sub
  ig_chunk = 256

  mesh = plsc.VectorSubcoreMesh(core_axis_name="c", subcore_axis_name="s")

  @functools.partial(
      pl.kernel,
      mesh=mesh,
      compiler_params=pltpu.CompilerParams(
          needs_layout_passes=False, use_tc_tiling_on_sc=False),
      out_type=(
          jax.ShapeDtypeStruct((B, D), jnp.float32),
          jax.ShapeDtypeStruct((B * D,), jnp.float32),
          jax.ShapeDtypeStruct((B * D,), jnp.float32),
      ),
      scratch_types=(
          pltpu.VMEM((ig_chunk,), jnp.int32),
          pltpu.VMEM((ig_chunk,), jnp.int32),
          pltpu.VMEM((ig_chunk, D), jnp.float32),
          pltpu.VMEM((bpw * L,), jnp.int32),
          pltpu.VMEM((1000 * D,), jnp.float32),
          pltpu.VMEM((3 * NB * D,), jnp.float32),
          pltpu.VMEM((32,), jnp.float32),
          pltpu.VMEM((3 * bpw,), jnp.float32),
          pltpu.VMEM((sub * D,), jnp.float32),
          pltpu.VMEM((sub * D,), jnp.float32),
          pltpu.SemaphoreType.DMA,
      ),
  )
  def k(item_id_h, genres_h, xs_h, bounds_h, table_h, gt_h, t3_h,
        iout_h, gout_h, cout_h,
        idx_a, idx_b, rows_v, gen_v, gt_v, t3_v, bnd_v, xs_v, gbuf, cbuf, sem):
    wid = lax.axis_index("s") * NC + lax.axis_index("c")
    base = wid * bpw

    pltpu.sync_copy(item_id_h.at[pl.ds(base, ig_chunk)], idx_a)
    cp_a = pltpu.async_copy(table_h.at[idx_a], rows_v, sem)
    pltpu.sync_copy(item_id_h.at[pl.ds(base + ig_chunk, ig_chunk)], idx_b)
    pltpu.sync_copy(genres_h.at[pl.ds(base * L, bpw * L)], gen_v)
    pltpu.sync_copy(gt_h, gt_v)
    pltpu.sync_copy(t3_h, t3_v)
    pltpu.sync_copy(bounds_h, bnd_v)
    for t in range(3):
      pltpu.sync_copy(xs_h.at[pl.ds(t * B + base, bpw)], xs_v.at[pl.ds(t * bpw, bpw)])

    zero = jnp.zeros((LANES,), jnp.float32)
    for j in range(D // LANES):
      gt_v[pl.ds(LANES * j, LANES)] = zero

    iota = lax.iota(jnp.int32, LANES)
    ba = bnd_v[pl.ds(0, LANES)]
    bb = bnd_v[pl.ds(LANES, LANES)]

    onev = jnp.ones((LANES,), jnp.float32)

    def row_body(i, sck):
      bi = sck * sub + i
      gbase = bi * L
      acc = [zero, zero, zero, zero]
      cntv = zero
      for l in range(L):
        gsp = plsc.load_gather(gen_v, [jnp.full((LANES,), gbase + l, jnp.int32)])
        cntv = cntv + jnp.where(gsp > 0, onev, zero)
        rb = gsp * D
        for j in range(D // LANES):
          acc[j] = acc[j] + plsc.load_gather(gt_v, [rb + (LANES * j) + iota])
      inv = onev / (cntv + jnp.float32(1e-8))
      for j in range(D // LANES):
        gbuf[pl.ds(i * D + LANES * j, LANES)] = acc[j] * inv
      cacc = [zero, zero, zero, zero]
      for t in range(3):
        xsp = plsc.load_gather(xs_v, [jnp.full((LANES,), t * bpw + bi, jnp.int32)])
        bidx = (plsc.all_reduce_population_count(ba < xsp)
                + plsc.all_reduce_population_count(bb < xsp))
        rb2 = bidx * D + (t * NB * D)
        for j in range(D // LANES):
          cacc[j] = cacc[j] + plsc.load_gather(t3_v, [rb2 + LANES * j + iota])
      for j in range(D // LANES):
        cbuf[pl.ds(i * D + LANES * j, LANES)] = cacc[j]
      return sck

    for sck in range(n_sub):
      lax.fori_loop(0, sub, row_body, sck)
      pltpu.sync_copy(gbuf, gout_h.at[pl.ds((base + sck * sub) * D, sub * D)])
      pltpu.sync_copy(cbuf, cout_h.at[pl.ds((base + sck * sub) * D, sub * D)])
      if sck == n_sub // 2 - 1:
        cp_a.wait()
        pltpu.sync_copy(rows_v, iout_h.at[pl.ds(base, ig_chunk)])
        cp_b = pltpu.async_copy(table_h.at[idx_b], rows_v, sem)
    cp_b.wait()
    pltpu.sync_copy(rows_v, iout_h.at[pl.ds(base + ig_chunk, ig_chunk)])

  return k(item_id, genres_flat, xs, bounds_pad, item_table, gt_flat, t3_flat)


def _mlp(i_emb, g_emb, c_emb, W1, bias3, W2, b2):
  B = i_emb.shape[0]
  blk = 2048

  def body(i_ref, g_ref, c_ref, w1_ref, bias_ref, w2_ref, b2_ref, o_ref):
    h = (jnp.dot(i_ref[...], w1_ref[0:D, :], preferred_element_type=jnp.float32)
         + jnp.dot(g_ref[...], w1_ref[D:2 * D, :], preferred_element_type=jnp.float32)
         + jnp.dot(c_ref[...], w1_ref[2 * D:3 * D, :], preferred_element_type=jnp.float32)
         + bias_ref[0:1, :])
    mu = jnp.mean(h, axis=1, keepdims=True)
    d = h - mu
    var = jnp.mean(d * d, axis=1, keepdims=True)
    hn = d * lax.rsqrt(var + 1e-5) * bias_ref[1:2, :] + bias_ref[2:3, :]
    hn = jnp.maximum(hn, 0.0)
    out = jnp.dot(hn, w2_ref[...], preferred_element_type=jnp.float32) + b2_ref[0:1, :]
    nrm = jnp.sqrt(jnp.sum(out * out, axis=1, keepdims=True))
    o_ref[...] = out / jnp.maximum(nrm, 1e-12)

  return pl.pallas_call(
      body,
      grid=(B // blk,),
      in_specs=[
          pl.BlockSpec((blk, D), lambda i: (i, 0)),
          pl.BlockSpec((blk, D), lambda i: (i, 0)),
          pl.BlockSpec((blk, D), lambda i: (i, 0)),
          pl.BlockSpec((3 * D, 2 * D), lambda i: (0, 0)),
          pl.BlockSpec((4, 2 * D), lambda i: (0, 0)),
          pl.BlockSpec((2 * D, D), lambda i: (0, 0)),
          pl.BlockSpec((1, D), lambda i: (0, 0)),
      ],
      out_specs=pl.BlockSpec((blk, D), lambda i: (i, 0)),
      out_shape=jax.ShapeDtypeStruct((B, D), jnp.float32),
  )(i_emb, g_emb, c_emb, W1, bias3, W2, b2)


def kernel(item_id, tmdb_genres, release_year, avg_rating, revenue,
           item_table, genre_table, ry_table, ar_table, rev_table, bounds,
           W1, b1, g1, be1, W2, b2):
  B = item_id.shape[0]
  genres_flat = tmdb_genres.reshape(-1).astype(jnp.int32)
  xs = jnp.concatenate([release_year, avg_rating, revenue])
  bounds_pad = jnp.concatenate(
      [bounds, jnp.full((32 - bounds.shape[0],), 2.0, jnp.float32)])
  gt_flat = genre_table.reshape(-1)
  t3_flat = jnp.concatenate(
      [ry_table.reshape(-1), ar_table.reshape(-1), rev_table.reshape(-1)])

  i_emb, g_flat, c_flat = _sc_embed(
      item_id.astype(jnp.int32), genres_flat, xs, bounds_pad,
      item_table, gt_flat, t3_flat)
  g_emb = g_flat.reshape(B, D)
  c_emb = c_flat.reshape(B, D)

  bias3 = jnp.concatenate(
      [b1[None, :], g1[None, :], be1[None, :],
       jnp.zeros((1, 2 * D), jnp.float32)], axis=0)
  return _mlp(i_emb, g_emb, c_emb, W1, bias3, W2, b2[None, :])
